# baseline (speedup 1.0000x reference)
import jax
import jax.numpy as jnp
from jax import lax
from jax.experimental import pallas as pl

DIM = 32
N_EMBED = 1024
BT = 8192
MROWS = 40


def _vq_block(xt_ref, w_ref, wext_ref, qt_ref, difft_ref, ind_ref):
    xt = xt_ref[...]
    w = w_ref[...]
    e2 = jnp.sum(w * w, axis=0)
    sT = lax.dot_general(w, xt, (((0,), (0,)), ((), ())),
                         preferred_element_type=jnp.float32)
    sT = sT - 0.5 * e2[:, None]
    m = jnp.max(sT, axis=0)
    onehot = (sT == m[None, :]).astype(jnp.float32)
    qi = lax.dot_general(wext_ref[...], onehot, (((1,), (0,)), ((), ())),
                         preferred_element_type=jnp.float32)
    qt = qi[:DIM]
    ind = (qi[DIM] * 256.0 + qi[DIM + 1]).astype(jnp.int32)
    qt_ref[...] = xt + (qt - xt)
    difft_ref[...] = (qt - xt) ** 2
    ind_ref[...] = ind


def kernel(inputs, embed):
    n_tokens = inputs.shape[0]
    grid = (n_tokens // BT,)
    iota = lax.iota(jnp.float32, N_EMBED)
    w_ext = jnp.concatenate(
        [embed, jnp.floor(iota / 256.0)[None, :], jnp.mod(iota, 256.0)[None, :],
         jnp.zeros((MROWS - DIM - 2, N_EMBED), jnp.float32)], axis=0)
    qt, difft, ind = pl.pallas_call(
        _vq_block,
        grid=grid,
        in_specs=[
            pl.BlockSpec((DIM, BT), lambda i: (0, i)),
            pl.BlockSpec((DIM, N_EMBED), lambda i: (0, 0)),
            pl.BlockSpec((MROWS, N_EMBED), lambda i: (0, 0)),
        ],
        out_specs=[
            pl.BlockSpec((DIM, BT), lambda i: (0, i)),
            pl.BlockSpec((DIM, BT), lambda i: (0, i)),
            pl.BlockSpec((BT,), lambda i: (i,)),
        ],
        out_shape=[
            jax.ShapeDtypeStruct((DIM, n_tokens), jnp.float32),
            jax.ShapeDtypeStruct((DIM, n_tokens), jnp.float32),
            jax.ShapeDtypeStruct((n_tokens,), jnp.int32),
        ],
    )(inputs.T, embed, w_ext)
    return (qt.T, difft.T.reshape(n_tokens, DIM, 1), ind.reshape(n_tokens, 1))


# speedup vs baseline: 1.0601x; 1.0601x over previous
import jax
import jax.numpy as jnp
from jax import lax
from jax.experimental import pallas as pl

DIM = 32
N_EMBED = 1024
BT = 8192


def _vq_block(xt_ref, w_ref, qt_ref, difft_ref, ind_ref):
    xt = xt_ref[...]
    w = w_ref[...]
    e2 = jnp.sum(w * w, axis=0)
    sT = lax.dot_general(w, xt, (((0,), (0,)), ((), ())),
                         preferred_element_type=jnp.float32)
    sT = sT - 0.5 * e2[:, None]
    ind = jnp.argmax(sT, axis=0).astype(jnp.int32)
    onehot = (lax.broadcasted_iota(jnp.int32, (N_EMBED, BT), 0)
              == ind[None, :]).astype(jnp.float32)
    qt = lax.dot_general(w, onehot, (((1,), (0,)), ((), ())),
                         preferred_element_type=jnp.float32)
    qt_ref[...] = xt + (qt - xt)
    difft_ref[...] = (qt - xt) ** 2
    ind_ref[...] = ind


def kernel(inputs, embed):
    n_tokens = inputs.shape[0]
    grid = (n_tokens // BT,)
    qt, difft, ind = pl.pallas_call(
        _vq_block,
        grid=grid,
        in_specs=[
            pl.BlockSpec((DIM, BT), lambda i: (0, i)),
            pl.BlockSpec((DIM, N_EMBED), lambda i: (0, 0)),
        ],
        out_specs=[
            pl.BlockSpec((DIM, BT), lambda i: (0, i)),
            pl.BlockSpec((DIM, BT), lambda i: (0, i)),
            pl.BlockSpec((BT,), lambda i: (i,)),
        ],
        out_shape=[
            jax.ShapeDtypeStruct((DIM, n_tokens), jnp.float32),
            jax.ShapeDtypeStruct((DIM, n_tokens), jnp.float32),
            jax.ShapeDtypeStruct((n_tokens,), jnp.int32),
        ],
    )(inputs.T, embed)
    return (qt.T, difft.T.reshape(n_tokens, DIM, 1), ind.reshape(n_tokens, 1))
